# Initial kernel scaffold; baseline (speedup 1.0000x reference)
import jax
import jax.numpy as jnp
from jax.experimental import pallas as pl


def kernel(prev, x, edge_index, W, b, gamma, beta):
    raise NotImplementedError("write your pallas kernel here")


# speedup vs baseline: 12.1316x; 12.1316x over previous
import functools

import jax
import jax.numpy as jnp
from jax import lax
from jax.experimental import pallas as pl
from jax.experimental.pallas import tpu as pltpu
from jax.experimental.pallas import tpu_sc as plsc

_N = 10000
_E = 160000
_D = 256
_NPAD = 10240
_EP = 163840
_RPT = _NPAD // 16

_mesh = plsc.VectorSubcoreMesh(core_axis_name="c", subcore_axis_name="s")


@functools.partial(
    pl.kernel,
    out_type=jax.ShapeDtypeStruct((2, _NPAD), jnp.float32),
    mesh=_mesh,
    scratch_types=[
        pltpu.VMEM((40, 128), jnp.int32),
        pltpu.VMEM((128,), jnp.float32),
        pltpu.VMEM((_RPT,), jnp.float32),
        pltpu.VMEM_SHARED((_NPAD,), jnp.float32),
    ],
)
def _deg_kernel(dst_hbm, out_hbm, idx_v, ones_v, tmp_v, deg_sh):
    c = lax.axis_index("c")
    s = lax.axis_index("s")
    w = s * 2 + c
    pltpu.sync_copy(dst_hbm.at[w], idx_v)

    def _init(i, carry):
        ones_v[pl.ds(i * 16, 16)] = jnp.ones((16,), jnp.float32)
        return carry

    lax.fori_loop(0, 8, _init, None)

    def _zero(i, carry):
        tmp_v[pl.ds(i * 16, 16)] = jnp.zeros((16,), jnp.float32)
        return carry

    lax.fori_loop(0, _RPT // 16, _zero, None)
    pltpu.sync_copy(tmp_v, deg_sh.at[pl.ds(s * _RPT, _RPT)])
    plsc.subcore_barrier()

    def _scat(j, carry):
        pltpu.sync_copy(ones_v, deg_sh.at[idx_v.at[j]], add=True)
        return carry

    lax.fori_loop(0, 40, _scat, None)
    plsc.subcore_barrier()

    pltpu.sync_copy(deg_sh.at[pl.ds(s * _RPT, _RPT)], tmp_v)

    @pl.when(c == 0)
    def _():
        pltpu.sync_copy(tmp_v, out_hbm.at[0, pl.ds(s * _RPT, _RPT)])

    @pl.when(c == 1)
    def _():
        pltpu.sync_copy(tmp_v, out_hbm.at[1, pl.ds(s * _RPT, _RPT)])


@functools.partial(
    pl.kernel,
    out_type=jax.ShapeDtypeStruct((2, _NPAD, 128), jnp.float32),
    mesh=_mesh,
    scratch_types=[
        pltpu.VMEM((80, 128), jnp.int32),
        pltpu.VMEM((80, 128), jnp.int32),
        pltpu.VMEM((64, 128), jnp.float32),
        pltpu.VMEM((64, 128), jnp.float32),
        pltpu.VMEM_SHARED((_NPAD, 128), jnp.float32),
        pltpu.SemaphoreType.DMA,
        pltpu.SemaphoreType.DMA,
    ],
)
def _msg_kernel(hlo_hbm, hhi_hbm, src_hbm, dst_hbm, out_hbm,
                src_v, dst_v, rows0, rows1, acc_sh, sem0, sem1):
    c = lax.axis_index("c")
    s = lax.axis_index("s")
    pltpu.sync_copy(src_hbm.at[s], src_v)
    pltpu.sync_copy(dst_hbm.at[s], dst_v)

    def _zero(i, carry):
        for jj in range(8):
            rows0[i, pl.ds(jj * 16, 16)] = jnp.zeros((16,), jnp.float32)
        return carry

    lax.fori_loop(0, 64, _zero, None)
    for k in range(_RPT // 64):
        pltpu.sync_copy(rows0, acc_sh.at[pl.ds(s * _RPT + k * 64, 64)])
    plsc.subcore_barrier()

    bufs = ((rows0, sem0), (rows1, sem1))

    def _run(h_hbm):
        for bb in range(2):
            pltpu.make_async_copy(h_hbm.at[src_v.at[0, pl.ds(bb * 64, 64)]],
                                  bufs[bb][0], bufs[bb][1]).start()

        def _pair(r, carry):
            for bb in range(2):
                rb, sb = bufs[bb]
                half = pl.ds(bb * 64, 64)
                pltpu.make_async_copy(h_hbm.at[src_v.at[r, half]], rb,
                                      sb).wait()
                pltpu.sync_copy(rb, acc_sh.at[dst_v.at[r, half]], add=True)
                pltpu.make_async_copy(h_hbm.at[src_v.at[r + 1, half]], rb,
                                      sb).start()
            return carry

        lax.fori_loop(0, 79, _pair, None)
        for bb in range(2):
            rb, sb = bufs[bb]
            half = pl.ds(bb * 64, 64)
            pltpu.make_async_copy(h_hbm.at[src_v.at[79, half]], rb, sb).wait()
            pltpu.sync_copy(rb, acc_sh.at[dst_v.at[79, half]], add=True)

    @pl.when(c == 0)
    def _():
        _run(hlo_hbm)

    @pl.when(c == 1)
    def _():
        _run(hhi_hbm)

    plsc.subcore_barrier()

    def _writeback(out_view):
        for k in range(_RPT // 64):
            pltpu.sync_copy(acc_sh.at[pl.ds(s * _RPT + k * 64, 64)], rows0)
            pltpu.sync_copy(rows0, out_view.at[pl.ds(s * _RPT + k * 64, 64)])

    @pl.when(c == 0)
    def _():
        _writeback(out_hbm.at[0])

    @pl.when(c == 1)
    def _():
        _writeback(out_hbm.at[1])


def _mm_body(x_ref, w_ref, degp_ref, hlo_ref, hhi_ref, dinv_ref):
    degp = degp_ref[...]
    deg = degp[0] + degp[1] + 1.0
    dinv = lax.rsqrt(deg)
    h = jnp.dot(x_ref[...], w_ref[...], preferred_element_type=jnp.float32)
    hs = h * dinv
    hlo_ref[...] = hs[:, :128]
    hhi_ref[...] = hs[:, 128:]
    dinv_ref[...] = dinv


_mm_call = pl.pallas_call(
    _mm_body,
    grid=(10,),
    in_specs=[
        pl.BlockSpec((1000, _D), lambda i: (i, 0)),
        pl.BlockSpec((_D, _D), lambda i: (0, 0)),
        pl.BlockSpec((2, 1000, 1), lambda i: (0, i, 0)),
    ],
    out_specs=[
        pl.BlockSpec((1000, 128), lambda i: (i, 0)),
        pl.BlockSpec((1000, 128), lambda i: (i, 0)),
        pl.BlockSpec((1000, 1), lambda i: (i, 0)),
    ],
    out_shape=[
        jax.ShapeDtypeStruct((_N, 128), jnp.float32),
        jax.ShapeDtypeStruct((_N, 128), jnp.float32),
        jax.ShapeDtypeStruct((_N, 1), jnp.float32),
    ],
)


def _fin_body(prev_ref, alo_ref, ahi_ref, hlo_ref, hhi_ref, dinv_ref,
              b_ref, g_ref, bt_ref, o_ref):
    acc = jnp.concatenate([alo_ref[...], ahi_ref[...]], axis=1)
    hs = jnp.concatenate([hlo_ref[...], hhi_ref[...]], axis=1)
    out = prev_ref[...] + b_ref[...] + dinv_ref[...] * (acc + hs)
    m = jnp.mean(out, axis=0, keepdims=True)
    v = jnp.mean((out - m) * (out - m), axis=0, keepdims=True)
    y = (out - m) * lax.rsqrt(v + 1e-5) * g_ref[...] + bt_ref[...]
    o_ref[...] = jnp.maximum(y, 0.0)


_fin_call = pl.pallas_call(
    _fin_body,
    out_shape=jax.ShapeDtypeStruct((_N, _D), jnp.float32),
)


def kernel(prev, x, edge_index, W, b, gamma, beta):
    src = edge_index[0]
    dst = edge_index[1]
    padn = _EP - _E
    dstp = jnp.concatenate([dst, jnp.full((padn,), _N, jnp.int32)])
    srcp = jnp.concatenate([src, jnp.zeros((padn,), jnp.int32)])
    dst_deg = dstp.reshape(32, 40, 128)
    src_msg = srcp.reshape(16, 80, 128)
    dst_msg = dstp.reshape(16, 80, 128)

    degp = _deg_kernel(dst_deg)
    hlo, hhi, dinv = _mm_call(x, W, degp.reshape(2, _NPAD, 1))
    acc = _msg_kernel(hlo, hhi, src_msg, dst_msg)
    return _fin_call(prev, acc[0, :_N], acc[1, :_N], hlo, hhi, dinv,
                     b.reshape(1, _D), gamma.reshape(1, _D),
                     beta.reshape(1, _D))


# speedup vs baseline: 12.5618x; 1.0355x over previous
import functools

import jax
import jax.numpy as jnp
from jax import lax
from jax.experimental import pallas as pl
from jax.experimental.pallas import tpu as pltpu
from jax.experimental.pallas import tpu_sc as plsc

_N = 10000
_E = 160000
_D = 256
_NPAD = 10240
_EP = 163840
_RPT = _NPAD // 16

_mesh = plsc.VectorSubcoreMesh(core_axis_name="c", subcore_axis_name="s")


@functools.partial(
    pl.kernel,
    out_type=jax.ShapeDtypeStruct((2, _NPAD), jnp.float32),
    mesh=_mesh,
    scratch_types=[
        pltpu.VMEM((40, 128), jnp.int32),
        pltpu.VMEM((128,), jnp.float32),
        pltpu.VMEM((_RPT,), jnp.float32),
        pltpu.VMEM_SHARED((_NPAD,), jnp.float32),
    ],
)
def _deg_kernel(dst_hbm, out_hbm, idx_v, ones_v, tmp_v, deg_sh):
    c = lax.axis_index("c")
    s = lax.axis_index("s")
    w = s * 2 + c
    pltpu.sync_copy(dst_hbm.at[w], idx_v)

    def _init(i, carry):
        ones_v[pl.ds(i * 16, 16)] = jnp.ones((16,), jnp.float32)
        return carry

    lax.fori_loop(0, 8, _init, None)

    def _zero(i, carry):
        tmp_v[pl.ds(i * 16, 16)] = jnp.zeros((16,), jnp.float32)
        return carry

    lax.fori_loop(0, _RPT // 16, _zero, None)
    pltpu.sync_copy(tmp_v, deg_sh.at[pl.ds(s * _RPT, _RPT)])
    plsc.subcore_barrier()

    def _scat(j, carry):
        pltpu.sync_copy(ones_v, deg_sh.at[idx_v.at[j]], add=True)
        return carry

    lax.fori_loop(0, 40, _scat, None)
    plsc.subcore_barrier()

    pltpu.sync_copy(deg_sh.at[pl.ds(s * _RPT, _RPT)], tmp_v)

    @pl.when(c == 0)
    def _():
        pltpu.sync_copy(tmp_v, out_hbm.at[0, pl.ds(s * _RPT, _RPT)])

    @pl.when(c == 1)
    def _():
        pltpu.sync_copy(tmp_v, out_hbm.at[1, pl.ds(s * _RPT, _RPT)])


@functools.partial(
    pl.kernel,
    out_type=jax.ShapeDtypeStruct((2, _NPAD, 128), jnp.float32),
    mesh=_mesh,
    scratch_types=[
        pltpu.VMEM((80, 128), jnp.int32),
        pltpu.VMEM((80, 128), jnp.int32),
        pltpu.VMEM((64, 128), jnp.float32),
        pltpu.VMEM((64, 128), jnp.float32),
        pltpu.VMEM((64, 128), jnp.float32),
        pltpu.VMEM_SHARED((_NPAD, 128), jnp.float32),
        pltpu.SemaphoreType.DMA,
        pltpu.SemaphoreType.DMA,
        pltpu.SemaphoreType.DMA,
        pltpu.SemaphoreType.DMA,
        pltpu.SemaphoreType.DMA,
        pltpu.SemaphoreType.DMA,
    ],
)
def _msg_kernel(hlo_hbm, hhi_hbm, src_hbm, dst_hbm, out_hbm,
                src_v, dst_v, rows0, rows1, rows2, acc_sh,
                sem0, sem1, sem2, ssem0, ssem1, ssem2):
    c = lax.axis_index("c")
    s = lax.axis_index("s")
    pltpu.sync_copy(src_hbm.at[s], src_v)
    pltpu.sync_copy(dst_hbm.at[s], dst_v)

    def _zero(i, carry):
        for jj in range(8):
            rows0[i, pl.ds(jj * 16, 16)] = jnp.zeros((16,), jnp.float32)
        return carry

    lax.fori_loop(0, 64, _zero, None)
    for k in range(_RPT // 64):
        pltpu.sync_copy(rows0, acc_sh.at[pl.ds(s * _RPT + k * 64, 64)])
    plsc.subcore_barrier()

    RB = (rows0, rows1, rows2)
    G = (sem0, sem1, sem2)
    SCS = (ssem0, ssem1, ssem2)

    def _run(h_hbm):
        def _gather(row, half, b):
            return pltpu.make_async_copy(
                h_hbm.at[src_v.at[row, pl.ds(half * 64, 64)]], RB[b], G[b])

        def _sc_drain(b):
            pltpu.make_async_copy(RB[b], acc_sh.at[dst_v.at[0, pl.ds(0, 64)]],
                                  SCS[b]).wait()

        def _step(row, half, b, prow, phalf, pb, scw, pre):
            _gather(row, half, b).wait()
            pltpu.async_copy(RB[b],
                             acc_sh.at[dst_v.at[row, pl.ds(half * 64, 64)]],
                             SCS[b], add=True)
            if scw:
                _sc_drain(pb)
            if pre:
                _gather(prow, phalf, pb).start()

        _gather(0, 0, 0).start()
        _gather(0, 1, 1).start()
        _step(0, 0, 0, 1, 0, 2, False, True)

        def _six(k, carry):
            r3 = k * 3
            for off in range(6):
                j = 1 + off
                _step(r3 + j // 2, j % 2, j % 3,
                      r3 + (j + 2) // 2, (j + 2) % 2, (j + 2) % 3, True, True)
            return carry

        lax.fori_loop(0, 26, _six, None)
        _step(78, 1, 1, 79, 1, 0, True, True)
        _step(79, 0, 2, 0, 0, 0, False, False)
        _step(79, 1, 0, 0, 0, 0, False, False)
        for b in range(3):
            _sc_drain(b)

    @pl.when(c == 0)
    def _():
        _run(hlo_hbm)

    @pl.when(c == 1)
    def _():
        _run(hhi_hbm)

    plsc.subcore_barrier()

    def _writeback(out_view):
        for k in range(_RPT // 64):
            pltpu.sync_copy(acc_sh.at[pl.ds(s * _RPT + k * 64, 64)], rows0)
            pltpu.sync_copy(rows0, out_view.at[pl.ds(s * _RPT + k * 64, 64)])

    @pl.when(c == 0)
    def _():
        _writeback(out_hbm.at[0])

    @pl.when(c == 1)
    def _():
        _writeback(out_hbm.at[1])


def _mm_body(x_ref, w_ref, degp_ref, hlo_ref, hhi_ref, dinv_ref):
    degp = degp_ref[...]
    deg = degp[0] + degp[1] + 1.0
    dinv = lax.rsqrt(deg)
    h = jnp.dot(x_ref[...], w_ref[...], preferred_element_type=jnp.float32)
    hs = h * dinv
    hlo_ref[...] = hs[:, :128]
    hhi_ref[...] = hs[:, 128:]
    dinv_ref[...] = dinv


_mm_call = pl.pallas_call(
    _mm_body,
    grid=(10,),
    in_specs=[
        pl.BlockSpec((1000, _D), lambda i: (i, 0)),
        pl.BlockSpec((_D, _D), lambda i: (0, 0)),
        pl.BlockSpec((2, 1000, 1), lambda i: (0, i, 0)),
    ],
    out_specs=[
        pl.BlockSpec((1000, 128), lambda i: (i, 0)),
        pl.BlockSpec((1000, 128), lambda i: (i, 0)),
        pl.BlockSpec((1000, 1), lambda i: (i, 0)),
    ],
    out_shape=[
        jax.ShapeDtypeStruct((_N, 128), jnp.float32),
        jax.ShapeDtypeStruct((_N, 128), jnp.float32),
        jax.ShapeDtypeStruct((_N, 1), jnp.float32),
    ],
)


def _fin_body(prev_ref, alo_ref, ahi_ref, hlo_ref, hhi_ref, dinv_ref,
              b_ref, g_ref, bt_ref, o_ref):
    acc = jnp.concatenate([alo_ref[...], ahi_ref[...]], axis=1)
    hs = jnp.concatenate([hlo_ref[...], hhi_ref[...]], axis=1)
    out = prev_ref[...] + b_ref[...] + dinv_ref[...] * (acc + hs)
    m = jnp.mean(out, axis=0, keepdims=True)
    v = jnp.mean((out - m) * (out - m), axis=0, keepdims=True)
    y = (out - m) * lax.rsqrt(v + 1e-5) * g_ref[...] + bt_ref[...]
    o_ref[...] = jnp.maximum(y, 0.0)


_fin_call = pl.pallas_call(
    _fin_body,
    out_shape=jax.ShapeDtypeStruct((_N, _D), jnp.float32),
)


def kernel(prev, x, edge_index, W, b, gamma, beta):
    src = edge_index[0]
    dst = edge_index[1]
    padn = _EP - _E
    dstp = jnp.concatenate([dst, jnp.full((padn,), _N, jnp.int32)])
    srcp = jnp.concatenate([src, jnp.zeros((padn,), jnp.int32)])
    dst_deg = dstp.reshape(32, 40, 128)
    src_msg = srcp.reshape(16, 80, 128)
    dst_msg = dstp.reshape(16, 80, 128)

    degp = _deg_kernel(dst_deg)
    hlo, hhi, dinv = _mm_call(x, W, degp.reshape(2, _NPAD, 1))
    acc = _msg_kernel(hlo, hhi, src_msg, dst_msg)
    return _fin_call(prev, acc[0, :_N], acc[1, :_N], hlo, hhi, dinv,
                     b.reshape(1, _D), gamma.reshape(1, _D),
                     beta.reshape(1, _D))
